# baseline (speedup 1.0000x reference)
import functools

import jax
import jax.numpy as jnp
from jax import lax
from jax.experimental import pallas as pl
from jax.experimental.pallas import tpu as pltpu
from jax.experimental.pallas import tpu_sc as plsc

B, T, D = 16, 2048, 64
NC, NS, L = 2, 16, 16
NW = NC * NS
TPW = T // NW
DV = D // L
NR = B * TPW

_mesh = plsc.VectorSubcoreMesh(core_axis_name="c", subcore_axis_name="s")


@functools.partial(
    pl.kernel,
    mesh=_mesh,
    out_type=jax.ShapeDtypeStruct((B, T, D), jnp.float32),
    scratch_types=[
        pltpu.VMEM((NR,), jnp.int32),
        pltpu.VMEM((NR // 2, D), jnp.float32),
        pltpu.VMEM((TPW, D), jnp.float32),
        pltpu.SemaphoreType.DMA,
    ],
)
def _emb_kernel(x_hbm, tok_hbm, pos_hbm, out_hbm, idx_v, rows_v, pos_v, sem):
    wid = lax.axis_index("s") * NC + lax.axis_index("c")
    p0 = wid * TPW

    for b in range(B):
        pltpu.sync_copy(x_hbm.at[b, pl.ds(p0, TPW)], idx_v.at[pl.ds(b * TPW, TPW)])
    pltpu.sync_copy(pos_hbm.at[pl.ds(p0, TPW)], pos_v)

    HB = B // 2
    HR = NR // 2

    for g in range(2):
        def fire(jc, _, g=g):
            xv = idx_v[pl.ds(g * HR + jc * L, L)]
            for l in range(L):
                pltpu.async_copy(tok_hbm.at[xv[l]], rows_v.at[jc * L + l], sem)
            return _

        lax.fori_loop(0, HR // L, fire, None)

        pltpu.make_async_copy(out_hbm.at[0, pl.ds(0, HR)], rows_v, sem).wait()

        def add_pos(j, _):
            for c in range(DV):
                pv = pos_v[j, pl.ds(c * L, L)]
                for b in range(HB):
                    plsc.addupdate(rows_v.at[b * TPW + j, pl.ds(c * L, L)], pv)
            return _

        lax.fori_loop(0, TPW, add_pos, None)

        for b in range(HB):
            pltpu.sync_copy(rows_v.at[pl.ds(b * TPW, TPW)],
                            out_hbm.at[g * HB + b, pl.ds(p0, TPW)])


V = 1000000
TB = 512


def _xpose_body(t_ref, o_ref):
    o_ref[...] = t_ref[...].T


_xpose = pl.pallas_call(
    _xpose_body,
    grid=((V + TB - 1) // TB,),
    in_specs=[pl.BlockSpec((D, TB), lambda j: (0, j))],
    out_specs=pl.BlockSpec((TB, D), lambda j: (j, 0)),
    out_shape=jax.ShapeDtypeStruct((V, D), jnp.float32),
)


def kernel(x, token_table, pos_table):
    conv = _xpose(token_table.T)
    return _emb_kernel(x, conv, pos_table)


# speedup vs baseline: 3.2124x; 3.2124x over previous
import functools

import jax
import jax.numpy as jnp
from jax import lax
from jax.experimental import pallas as pl
from jax.experimental.pallas import tpu as pltpu
from jax.experimental.pallas import tpu_sc as plsc

B, T, D = 16, 2048, 64
V = 1000000
NC, NS, L = 2, 16, 16
NW = NC * NS
TPW = T // NW
DV = D // L
NR = B * TPW
TB = 4096

_mesh = plsc.VectorSubcoreMesh(core_axis_name="c", subcore_axis_name="s")


def _xpose_body(t_ref, o_ref):
    o_ref[...] = t_ref[...].T


_xpose = pl.pallas_call(
    _xpose_body,
    grid=((V + TB - 1) // TB,),
    in_specs=[pl.BlockSpec((D, TB), lambda j: (0, j))],
    out_specs=pl.BlockSpec((TB, D), lambda j: (j, 0)),
    out_shape=jax.ShapeDtypeStruct((V, D), jnp.float32),
)


@functools.partial(
    pl.kernel,
    mesh=_mesh,
    out_type=jax.ShapeDtypeStruct((B, T, D), jnp.float32),
    scratch_types=[
        pltpu.VMEM((NR,), jnp.int32),
        pltpu.VMEM((NR // 2, D), jnp.float32),
        pltpu.VMEM((TPW, D), jnp.float32),
        pltpu.SemaphoreType.DMA,
    ],
)
def _emb_kernel(x_hbm, tok_hbm, pos_hbm, out_hbm, idx_v, rows_v, pos_v, sem):
    wid = lax.axis_index("s") * NC + lax.axis_index("c")
    p0 = wid * TPW

    for b in range(B):
        pltpu.sync_copy(x_hbm.at[b, pl.ds(p0, TPW)], idx_v.at[pl.ds(b * TPW, TPW)])
    pltpu.sync_copy(pos_hbm.at[pl.ds(p0, TPW)], pos_v)

    HB = B // 2
    HR = NR // 2

    for g in range(2):
        def fire(jc, _, g=g):
            xv = idx_v[pl.ds(g * HR + jc * L, L)]
            for l in range(L):
                pltpu.async_copy(tok_hbm.at[xv[l]], rows_v.at[jc * L + l], sem)
            return _

        lax.fori_loop(0, HR // L, fire, None)

        pltpu.make_async_copy(out_hbm.at[0, pl.ds(0, HR)], rows_v, sem).wait()

        def add_pos(j, _):
            for c in range(DV):
                pv = pos_v[j, pl.ds(c * L, L)]
                for b in range(HB):
                    plsc.addupdate(rows_v.at[b * TPW + j, pl.ds(c * L, L)], pv)
            return _

        lax.fori_loop(0, TPW, add_pos, None)

        for b in range(HB):
            pltpu.sync_copy(rows_v.at[pl.ds(b * TPW, TPW)],
                            out_hbm.at[g * HB + b, pl.ds(p0, TPW)])


def kernel(x, token_table, pos_table):
    conv = _xpose(token_table.T)
    return _emb_kernel(x, conv, pos_table)


# speedup vs baseline: 3.6650x; 1.1409x over previous
import functools

import jax
import jax.numpy as jnp
from jax import lax
from jax.experimental import pallas as pl
from jax.experimental.pallas import tpu as pltpu
from jax.experimental.pallas import tpu_sc as plsc

B, T, D = 16, 2048, 64
V = 1000000
NC, NS, L = 2, 16, 16
NW = NC * NS
TPW = T // NW
DV = D // L
BG = 4
TB = 4096
NB = 123
O2 = 122 * TB
CUT = NB * TB
VMAIN = 244 * TB
NTAIL = V - VMAIN

_mesh = plsc.VectorSubcoreMesh(core_axis_name="c", subcore_axis_name="s")


def _xpose_body(a_ref, b_ref, o_ref):
    o_ref[...] = jnp.concatenate([a_ref[...].T, b_ref[...].T], axis=1)


_xpose = pl.pallas_call(
    _xpose_body,
    grid=(NB,),
    in_specs=[
        pl.BlockSpec((D, TB), lambda j: (0, j)),
        pl.BlockSpec((D, TB), lambda j: (0, j + 122)),
    ],
    out_specs=pl.BlockSpec((TB, 2 * D), lambda j: (j, 0)),
    out_shape=jax.ShapeDtypeStruct((CUT, 2 * D), jnp.float32),
)


@functools.partial(
    pl.kernel,
    mesh=_mesh,
    out_type=jax.ShapeDtypeStruct((B, T, D), jnp.float32),
    scratch_types=[
        pltpu.VMEM((B * TPW + L,), jnp.int32),
        pltpu.VMEM((B, TPW), jnp.int32),
        pltpu.VMEM((BG, TPW, 2 * D), jnp.float32),
        pltpu.VMEM((BG, TPW, D), jnp.float32),
        pltpu.VMEM((TPW, D), jnp.float32),
        pltpu.SemaphoreType.DMA,
    ],
)
def _emb_kernel(x_hbm, tokw_hbm, pos_hbm, out_hbm,
                idx_v, xe_v, wide_v, rows_v, pos_v, sem):
    wid = lax.axis_index("s") * NC + lax.axis_index("c")
    p0 = wid * TPW

    for b in range(B):
        pltpu.sync_copy(x_hbm.at[b, pl.ds(p0, TPW)], idx_v.at[pl.ds(b * TPW, TPW)])
    pltpu.sync_copy(pos_hbm.at[pl.ds(p0, TPW)], pos_v)

    for b in range(B):
        for c in range(TPW // L):
            v = idx_v[pl.ds(b * TPW + c * L, L)]
            xe_v[b, pl.ds(c * L, L)] = jnp.where(v >= CUT, v - O2, v)

    for g in range(B // BG):
        copies = [
            pltpu.async_copy(
                tokw_hbm.at[xe_v.at[g * BG + b]], wide_v.at[b], sem)
            for b in range(BG)
        ]
        for cp in copies:
            cp.wait()

        def sel_add(j, _, g=g):
            pvs = [pos_v[j, pl.ds(c * L, L)] for c in range(DV)]
            for b in range(BG):
                xv = idx_v[pl.ds((g * BG + b) * TPW + j, L)]
                off = jnp.where(xv[0] >= CUT, D, 0)
                for c in range(DV):
                    lo = wide_v[b, j, pl.ds(off + c * L, L)]
                    rows_v[b, j, pl.ds(c * L, L)] = lo + pvs[c]
            return _

        lax.fori_loop(0, TPW, sel_add, None)

        for b in range(BG):
            pltpu.sync_copy(rows_v.at[b], out_hbm.at[g * BG + b, pl.ds(p0, TPW)])


def kernel(x, token_table, pos_table):
    tokw = _xpose(token_table.T, token_table.T)
    return _emb_kernel(x, tokw, pos_table)


# speedup vs baseline: 4.0475x; 1.1044x over previous
import functools

import jax
import jax.numpy as jnp
from jax import lax
from jax.experimental import pallas as pl
from jax.experimental.pallas import tpu as pltpu
from jax.experimental.pallas import tpu_sc as plsc

B, T, D = 16, 2048, 64
V = 1000000
NC, NS, L = 2, 16, 16
NW = NC * NS
TPW = T // NW
DV = D // L
BG = 4
TB = 8192
NB = 62
O2 = 61 * TB
CUT = NB * TB

_mesh = plsc.VectorSubcoreMesh(core_axis_name="c", subcore_axis_name="s")


def _xpose_body(a_ref, b_ref, o_ref):
    o_ref[...] = jnp.concatenate([a_ref[...].T, b_ref[...].T], axis=1)


_xpose = pl.pallas_call(
    _xpose_body,
    grid=(NB,),
    in_specs=[
        pl.BlockSpec((D, TB), lambda j: (0, j)),
        pl.BlockSpec((D, TB), lambda j: (0, j + 61)),
    ],
    out_specs=pl.BlockSpec((TB, 2 * D), lambda j: (j, 0)),
    out_shape=jax.ShapeDtypeStruct((CUT, 2 * D), jnp.float32),
)


@functools.partial(
    pl.kernel,
    mesh=_mesh,
    out_type=jax.ShapeDtypeStruct((B, T, D), jnp.float32),
    scratch_types=[
        pltpu.VMEM((B * TPW + L,), jnp.int32),
        pltpu.VMEM((B, TPW), jnp.int32),
        pltpu.VMEM((BG, TPW, 2 * D), jnp.float32),
        pltpu.VMEM((BG, TPW, D), jnp.float32),
        pltpu.VMEM((TPW, D), jnp.float32),
        pltpu.SemaphoreType.DMA,
    ],
)
def _emb_kernel(x_hbm, tokw_hbm, pos_hbm, out_hbm,
                idx_v, xe_v, wide_v, rows_v, pos_v, sem):
    wid = lax.axis_index("s") * NC + lax.axis_index("c")
    p0 = wid * TPW

    for b in range(B):
        pltpu.sync_copy(x_hbm.at[b, pl.ds(p0, TPW)], idx_v.at[pl.ds(b * TPW, TPW)])
    pltpu.sync_copy(pos_hbm.at[pl.ds(p0, TPW)], pos_v)

    for b in range(B):
        for c in range(TPW // L):
            v = idx_v[pl.ds(b * TPW + c * L, L)]
            xe_v[b, pl.ds(c * L, L)] = jnp.where(v >= CUT, v - O2, v)

    for g in range(B // BG):
        copies = [
            pltpu.async_copy(
                tokw_hbm.at[xe_v.at[g * BG + b]], wide_v.at[b], sem)
            for b in range(BG)
        ]
        for cp in copies:
            cp.wait()

        def sel_add(j, _, g=g):
            pvs = [pos_v[j, pl.ds(c * L, L)] for c in range(DV)]
            for b in range(BG):
                xv = idx_v[pl.ds((g * BG + b) * TPW + j, L)]
                off = jnp.where(xv[0] >= CUT, D, 0)
                for c in range(DV):
                    lo = wide_v[b, j, pl.ds(off + c * L, L)]
                    rows_v[b, j, pl.ds(c * L, L)] = lo + pvs[c]
            return _

        lax.fori_loop(0, TPW, sel_add, None)

        for b in range(BG):
            pltpu.sync_copy(rows_v.at[b], out_hbm.at[g * BG + b, pl.ds(p0, TPW)])


def kernel(x, token_table, pos_table):
    tokw = _xpose(token_table.T, token_table.T)
    return _emb_kernel(x, tokw, pos_table)


# speedup vs baseline: 4.2391x; 1.0473x over previous
import functools

import jax
import jax.numpy as jnp
from jax import lax
from jax.experimental import pallas as pl
from jax.experimental.pallas import tpu as pltpu
from jax.experimental.pallas import tpu_sc as plsc

B, T, D = 16, 2048, 64
V = 1000000
NC, NS, L = 2, 16, 16
NW = NC * NS
TPW = T // NW
DV = D // L
BG = 4
TB = 16384
NB = 31
O2 = 31 * TB
CUT = NB * TB

_mesh = plsc.VectorSubcoreMesh(core_axis_name="c", subcore_axis_name="s")


def _xpose_body(a_ref, b_ref, o_ref):
    o_ref[...] = jnp.concatenate([a_ref[...].T, b_ref[...].T], axis=1)


_xpose = pl.pallas_call(
    _xpose_body,
    grid=(NB,),
    in_specs=[
        pl.BlockSpec((D, TB), lambda j: (0, j)),
        pl.BlockSpec((D, TB), lambda j: (0, j + 31)),
    ],
    out_specs=pl.BlockSpec((TB, 2 * D), lambda j: (j, 0)),
    out_shape=jax.ShapeDtypeStruct((CUT, 2 * D), jnp.float32),
)


@functools.partial(
    pl.kernel,
    mesh=_mesh,
    out_type=jax.ShapeDtypeStruct((B, T, D), jnp.float32),
    scratch_types=[
        pltpu.VMEM((B * TPW + L,), jnp.int32),
        pltpu.VMEM((B, TPW), jnp.int32),
        pltpu.VMEM((BG, TPW, 2 * D), jnp.float32),
        pltpu.VMEM((BG, TPW, D), jnp.float32),
        pltpu.VMEM((TPW, D), jnp.float32),
        pltpu.SemaphoreType.DMA,
    ],
)
def _emb_kernel(x_hbm, tokw_hbm, pos_hbm, out_hbm,
                idx_v, xe_v, wide_v, rows_v, pos_v, sem):
    wid = lax.axis_index("s") * NC + lax.axis_index("c")
    p0 = wid * TPW

    for b in range(B):
        pltpu.sync_copy(x_hbm.at[b, pl.ds(p0, TPW)], idx_v.at[pl.ds(b * TPW, TPW)])
    pltpu.sync_copy(pos_hbm.at[pl.ds(p0, TPW)], pos_v)

    for b in range(B):
        for c in range(TPW // L):
            v = idx_v[pl.ds(b * TPW + c * L, L)]
            xe_v[b, pl.ds(c * L, L)] = jnp.where(v >= CUT, v - O2, v)

    for g in range(B // BG):
        copies = [
            pltpu.async_copy(
                tokw_hbm.at[xe_v.at[g * BG + b]], wide_v.at[b], sem)
            for b in range(BG)
        ]
        for cp in copies:
            cp.wait()

        def sel_add(j, _, g=g):
            pvs = [pos_v[j, pl.ds(c * L, L)] for c in range(DV)]
            for b in range(BG):
                xv = idx_v[pl.ds((g * BG + b) * TPW + j, L)]
                off = jnp.where(xv[0] >= CUT, D, 0)
                for c in range(DV):
                    lo = wide_v[b, j, pl.ds(off + c * L, L)]
                    rows_v[b, j, pl.ds(c * L, L)] = lo + pvs[c]
            return _

        lax.fori_loop(0, TPW, sel_add, None)

        for b in range(BG):
            pltpu.sync_copy(rows_v.at[b], out_hbm.at[g * BG + b, pl.ds(p0, TPW)])


def kernel(x, token_table, pos_table):
    tokw = _xpose(token_table.T, token_table.T)
    return _emb_kernel(x, tokw, pos_table)
